# Initial kernel scaffold; baseline (speedup 1.0000x reference)
import functools
import jax
import jax.numpy as jnp
from jax import lax
from jax.experimental import pallas as pl
from jax.experimental.pallas import tpu as pltpu
from jax.experimental.pallas import tpu_sc as plsc

N_U = 4000
N_I = 6000
N = N_U + N_I
E = 320000
D = 128
F = 4
FD = D // F

NC, NS = 2, 16
NW = NC * NS
CH = 128
EPT = 10240
CHUNKS = EPT // CH
E_PAD = NW * EPT
N_ACC = 10112
ZR = N_ACC // NS

_MESH = plsc.VectorSubcoreMesh(
    core_axis_name="c", subcore_axis_name="s", num_cores=NC, num_subcores=NS)


def _wid():
    return lax.axis_index("c") * NS + lax.axis_index("s")


@functools.partial(
    pl.kernel,
    out_type=jax.ShapeDtypeStruct((NC, N_ACC), jnp.float32),
    mesh=_MESH,
    scratch_types=[
        pltpu.VMEM((CHUNKS, CH), jnp.int32),
        pltpu.VMEM((CH,), jnp.float32),
        pltpu.VMEM_SHARED((N_ACC,), jnp.float32),
    ],
)
def _sc_deg(h2_hbm, z1_hbm, out_hbm, hidx, ones, acc):
    c = lax.axis_index("c")
    s = lax.axis_index("s")
    w = _wid()
    pltpu.sync_copy(h2_hbm.at[pl.ds(w * CHUNKS, CHUNKS)], hidx)
    for i in range(CH // 16):
        ones[pl.ds(16 * i, 16)] = jnp.ones((16,), jnp.float32)
    pltpu.sync_copy(z1_hbm.at[pl.ds(s * ZR, ZR)], acc.at[pl.ds(s * ZR, ZR)])
    plsc.subcore_barrier()

    def chunk(j, carry):
        pltpu.sync_copy(ones, acc.at[hidx.at[j]], add=True)
        return carry

    lax.fori_loop(0, CHUNKS, chunk, 0)
    plsc.subcore_barrier()

    @pl.when(s == 0)
    def _():
        pltpu.sync_copy(acc, out_hbm.at[c])


@functools.partial(
    pl.kernel,
    out_type=jax.ShapeDtypeStruct((NC, N_ACC, D), jnp.float32),
    mesh=_MESH,
    scratch_types=[
        pltpu.VMEM((CHUNKS, CH), jnp.int32),
        pltpu.VMEM((CHUNKS, CH), jnp.int32),
        pltpu.VMEM((CH, D), jnp.float32),
        pltpu.VMEM_SHARED((N_ACC, D), jnp.float32),
    ],
)
def _sc_pass0(y0_hbm, t2_hbm, h2_hbm, znd_hbm, out_hbm, tidx, hidx, rows, xacc):
    c = lax.axis_index("c")
    s = lax.axis_index("s")
    w = _wid()
    pltpu.sync_copy(t2_hbm.at[pl.ds(w * CHUNKS, CHUNKS)], tidx)
    pltpu.sync_copy(h2_hbm.at[pl.ds(w * CHUNKS, CHUNKS)], hidx)
    pltpu.sync_copy(znd_hbm.at[pl.ds(s * ZR, ZR)], xacc.at[pl.ds(s * ZR, ZR)])
    plsc.subcore_barrier()

    def chunk(j, carry):
        pltpu.sync_copy(y0_hbm.at[tidx.at[j]], rows)
        pltpu.sync_copy(rows, xacc.at[hidx.at[j]], add=True)
        return carry

    lax.fori_loop(0, CHUNKS, chunk, 0)
    plsc.subcore_barrier()
    pltpu.sync_copy(xacc.at[pl.ds(s * ZR, ZR)], out_hbm.at[c, pl.ds(s * ZR, ZR)])


@functools.partial(
    pl.kernel,
    out_type=(
        jax.ShapeDtypeStruct((F, E_PAD), jnp.float32),
        jax.ShapeDtypeStruct((NC, F, N_ACC), jnp.float32),
    ),
    mesh=_MESH,
    scratch_types=[
        pltpu.VMEM((CHUNKS, CH), jnp.int32),
        pltpu.VMEM((CHUNKS, CH), jnp.int32),
        pltpu.VMEM((CH, D), jnp.float32),
        pltpu.VMEM((CH, D), jnp.float32),
        pltpu.VMEM((F, CH), jnp.float32),
        pltpu.VMEM((F, EPT), jnp.float32),
        pltpu.VMEM_SHARED((N_ACC,), jnp.float32),
        pltpu.VMEM_SHARED((N_ACC,), jnp.float32),
        pltpu.VMEM_SHARED((N_ACC,), jnp.float32),
        pltpu.VMEM_SHARED((N_ACC,), jnp.float32),
    ],
)
def _sc_route(xn_hbm, tn_hbm, h2_hbm, t2_hbm, z1_hbm, scores_hbm, rsum_hbm,
              hidx, tidx, abuf, bbuf, dots, sbuf, r0, r1, r2, r3):
    c = lax.axis_index("c")
    s = lax.axis_index("s")
    w = _wid()
    base = w * EPT
    raccs = (r0, r1, r2, r3)
    pltpu.sync_copy(h2_hbm.at[pl.ds(w * CHUNKS, CHUNKS)], hidx)
    pltpu.sync_copy(t2_hbm.at[pl.ds(w * CHUNKS, CHUNKS)], tidx)
    for i in range(F):
        pltpu.sync_copy(z1_hbm.at[pl.ds(s * ZR, ZR)],
                        raccs[i].at[pl.ds(s * ZR, ZR)])
    plsc.subcore_barrier()

    def chunk(j, carry):
        pltpu.sync_copy(xn_hbm.at[hidx.at[j]], abuf)
        pltpu.sync_copy(tn_hbm.at[tidx.at[j]], bbuf)

        def edges(e4, carry2):
            for u in range(4):
                e = e4 * 4 + u
                for i in range(F):
                    a0 = abuf[e, pl.ds(FD * i, 16)]
                    a1 = abuf[e, pl.ds(FD * i + 16, 16)]
                    b0 = bbuf[e, pl.ds(FD * i, 16)]
                    b1 = bbuf[e, pl.ds(FD * i + 16, 16)]
                    dots[i, e] = jnp.sum(a0 * b0 + a1 * b1)
            return carry2

        lax.fori_loop(0, CH // 4, edges, 0)
        for g in range(CH // 16):
            v = [dots[i, pl.ds(16 * g, 16)] for i in range(F)]
            m = jnp.maximum(jnp.maximum(v[0], v[1]), jnp.maximum(v[2], v[3]))
            ex = [jnp.exp(x - m) for x in v]
            inv = 1.0 / (ex[0] + ex[1] + ex[2] + ex[3])
            for i in range(F):
                sbuf[i, pl.ds(j * CH + 16 * g, 16)] = ex[i] * inv
        for i in range(F):
            pltpu.sync_copy(sbuf.at[i, pl.ds(j * CH, CH)],
                            raccs[i].at[hidx.at[j]], add=True)
        return carry

    lax.fori_loop(0, CHUNKS, chunk, 0)
    for i in range(F):
        pltpu.sync_copy(sbuf.at[i], scores_hbm.at[i, pl.ds(base, EPT)])
    plsc.subcore_barrier()
    for i in range(F):
        pltpu.sync_copy(raccs[i].at[pl.ds(s * ZR, ZR)],
                        rsum_hbm.at[c, i, pl.ds(s * ZR, ZR)])


@functools.partial(
    pl.kernel,
    out_type=jax.ShapeDtypeStruct((NC, N_ACC, D), jnp.float32),
    mesh=_MESH,
    scratch_types=[
        pltpu.VMEM((CHUNKS, CH), jnp.int32),
        pltpu.VMEM((CHUNKS, CH), jnp.int32),
        pltpu.VMEM((F, EPT), jnp.float32),
        pltpu.VMEM((CH, D), jnp.float32),
        pltpu.VMEM_SHARED((N_ACC, D), jnp.float32),
    ],
)
def _sc_pass1(y1_hbm, sc_hbm, t2_hbm, h2_hbm, znd_hbm, out_hbm,
              tidx, hidx, sbuf, rows, xacc):
    c = lax.axis_index("c")
    s = lax.axis_index("s")
    w = _wid()
    base = w * EPT
    pltpu.sync_copy(t2_hbm.at[pl.ds(w * CHUNKS, CHUNKS)], tidx)
    pltpu.sync_copy(h2_hbm.at[pl.ds(w * CHUNKS, CHUNKS)], hidx)
    for i in range(F):
        pltpu.sync_copy(sc_hbm.at[i, pl.ds(base, EPT)], sbuf.at[i])
    pltpu.sync_copy(znd_hbm.at[pl.ds(s * ZR, ZR)], xacc.at[pl.ds(s * ZR, ZR)])
    plsc.subcore_barrier()

    def chunk(j, carry):
        pltpu.sync_copy(y1_hbm.at[tidx.at[j]], rows)

        def edges(e4, carry2):
            for u in range(4):
                e = e4 * 4 + u
                for i in range(F):
                    sv = sbuf[i, j * CH + e]
                    for k in (2 * i, 2 * i + 1):
                        rows[e, pl.ds(16 * k, 16)] = rows[e, pl.ds(16 * k, 16)] * sv
            return carry2

        lax.fori_loop(0, CH // 4, edges, 0)
        pltpu.sync_copy(rows, xacc.at[hidx.at[j]], add=True)
        return carry

    lax.fori_loop(0, CHUNKS, chunk, 0)
    plsc.subcore_barrier()
    pltpu.sync_copy(xacc.at[pl.ds(s * ZR, ZR)], out_hbm.at[c, pl.ds(s * ZR, ZR)])


def _slicenorm(x):
    outs = []
    for i in range(F):
        sl = x[:, i * FD:(i + 1) * FD]
        n = jnp.sqrt(jnp.sum(sl * sl, axis=1, keepdims=True))
        outs.append(sl / jnp.maximum(n, 1e-12))
    return jnp.concatenate(outs, axis=1)


def _tc_prep_body(ego_ref, deg_ref, y0_ref, tn_ref):
    ego = ego_ref[...]
    d0 = lax.rsqrt(0.25 * deg_ref[...] + 1e-12)
    y0_ref[...] = ego * d0
    tn_ref[...] = _slicenorm(jnp.tanh(ego))


def _tc_mid0_body(x0p_ref, xn_ref):
    r = x0p_ref[0] + x0p_ref[1]
    rowid = lax.broadcasted_iota(jnp.int32, (N_ACC, D), 0)
    r = jnp.where(rowid < N, r, 0.0)
    xn_ref[...] = _slicenorm(r)


def _tc_y1_body(ego_ref, rse_ref, y1_ref, d1e_ref):
    d1e = lax.rsqrt(rse_ref[...] + 1e-12)
    d1e_ref[...] = d1e
    y1_ref[...] = ego_ref[...] * d1e


def _tc_final_body(ego_ref, x1p_ref, d1e_ref, out_ref):
    x1 = (x1p_ref[0] + x1p_ref[1]) * d1e_ref[...]
    out_ref[...] = 0.5 * (ego_ref[...] + x1)


_f32 = jnp.float32


@jax.jit
def kernel(user_embedding, item_embedding, all_h_list, all_t_list):
    ego = jnp.concatenate(
        [user_embedding, item_embedding,
         jnp.zeros((N_ACC - N, D), _f32)], axis=0)
    pad = jnp.full((E_PAD - E,), N, jnp.int32)
    h2 = jnp.concatenate([all_h_list.astype(jnp.int32), pad]).reshape(-1, CH)
    t2 = jnp.concatenate([all_t_list.astype(jnp.int32), pad]).reshape(-1, CH)
    znd = jnp.zeros((N_ACC, D), _f32)
    z1 = jnp.zeros((N_ACC,), _f32)

    degp = _sc_deg(h2, z1)
    deg = (degp[0] + degp[1])[:, None]

    y0, tn = pl.pallas_call(
        _tc_prep_body,
        out_shape=(jax.ShapeDtypeStruct((N_ACC, D), _f32),
                   jax.ShapeDtypeStruct((N_ACC, D), _f32)))(ego, deg)

    x0p = _sc_pass0(y0, t2, h2, znd)

    xn0 = pl.pallas_call(
        _tc_mid0_body,
        out_shape=jax.ShapeDtypeStruct((N_ACC, D), _f32))(x0p)

    scores, rsump = _sc_route(xn0, tn, h2, t2, z1)
    rs = rsump[0] + rsump[1]
    rse = jnp.repeat(rs.T, FD, axis=1)

    y1, d1e = pl.pallas_call(
        _tc_y1_body,
        out_shape=(jax.ShapeDtypeStruct((N_ACC, D), _f32),
                   jax.ShapeDtypeStruct((N_ACC, D), _f32)))(ego, rse)

    x1p = _sc_pass1(y1, scores, t2, h2, znd)

    out = pl.pallas_call(
        _tc_final_body,
        out_shape=jax.ShapeDtypeStruct((N_ACC, D), _f32))(ego, x1p, d1e)
    return out[:N_U], out[N_U:N]


# speedup vs baseline: 8.0704x; 8.0704x over previous
import functools
import jax
import jax.numpy as jnp
from jax import lax
from jax.experimental import pallas as pl
from jax.experimental.pallas import tpu as pltpu
from jax.experimental.pallas import tpu_sc as plsc

N_U = 4000
N_I = 6000
N = N_U + N_I
E = 320000
D = 128
F = 4
FD = D // F

NC, NS = 2, 16
NW = NC * NS
CH = 128
EPT = 10240
CHUNKS = EPT // CH
E_PAD = NW * EPT
G8 = 8
GC = CHUNKS // G8
N_ACC = 10240
ZR = N_ACC // NS
ZQ = ZR // CH

_BITREV = (0, 8, 4, 12, 2, 10, 6, 14, 1, 9, 5, 13, 3, 11, 7, 15)

_MESH = plsc.VectorSubcoreMesh(
    core_axis_name="c", subcore_axis_name="s", num_cores=NC, num_subcores=NS)


def _wid():
    return lax.axis_index("c") * NS + lax.axis_index("s")


def _zero_rows(rows):
    def zrow(r, carry):
        for k in range(D // 16):
            rows[r, pl.ds(16 * k, 16)] = jnp.zeros((16,), jnp.float32)
        return carry

    lax.fori_loop(0, CH, zrow, 0)


@functools.partial(
    pl.kernel,
    out_type=jax.ShapeDtypeStruct((NC, N_ACC), jnp.float32),
    mesh=_MESH,
    compiler_params=pltpu.CompilerParams(needs_layout_passes=False),
    scratch_types=[
        pltpu.VMEM((CHUNKS, CH), jnp.int32),
        pltpu.VMEM((CH,), jnp.float32),
        pltpu.VMEM((ZR,), jnp.float32),
        pltpu.VMEM_SHARED((N_ACC,), jnp.float32),
    ],
)
def _sc_deg(h2_hbm, out_hbm, hidx, ones, stage, acc):
    c = lax.axis_index("c")
    s = lax.axis_index("s")
    w = _wid()
    pltpu.sync_copy(h2_hbm.at[pl.ds(w * CHUNKS, CHUNKS)], hidx)
    for i in range(CH // 16):
        ones[pl.ds(16 * i, 16)] = jnp.ones((16,), jnp.float32)

    def zrow(r, carry):
        stage[pl.ds(16 * r, 16)] = jnp.zeros((16,), jnp.float32)
        return carry

    lax.fori_loop(0, ZR // 16, zrow, 0)
    pltpu.sync_copy(stage, acc.at[pl.ds(s * ZR, ZR)])
    plsc.subcore_barrier()

    def chunk(j, carry):
        pltpu.sync_copy(ones, acc.at[hidx.at[j]], add=True)
        return carry

    lax.fori_loop(0, CHUNKS, chunk, 0)
    plsc.subcore_barrier()
    pltpu.sync_copy(acc.at[pl.ds(s * ZR, ZR)], stage)
    pltpu.sync_copy(stage, out_hbm.at[c, pl.ds(s * ZR, ZR)])


@functools.partial(
    pl.kernel,
    out_type=jax.ShapeDtypeStruct((NC, N_ACC, D), jnp.float32),
    mesh=_MESH,
    compiler_params=pltpu.CompilerParams(needs_layout_passes=False),
    scratch_types=[
        pltpu.VMEM((4, 2, CH), jnp.int32),
        pltpu.VMEM((2, CH, D), jnp.float32),
        pltpu.VMEM_SHARED((N_ACC, D), jnp.float32),
        pltpu.SemaphoreType.DMA((4,)),
        pltpu.SemaphoreType.DMA((2,)),
        pltpu.SemaphoreType.DMA((2,)),
    ],
)
def _sc_pass0(y0_hbm, ht_hbm, out_hbm, ht, rows, xacc, isem, gsem, ssem):
    c = lax.axis_index("c")
    s = lax.axis_index("s")
    w = _wid()
    hbase = w * CHUNKS
    _zero_rows(rows.at[0])
    for q in range(ZQ):
        pltpu.sync_copy(rows.at[0], xacc.at[pl.ds(s * ZR + q * CH, CH)])
    plsc.subcore_barrier()

    pltpu.async_copy(ht_hbm.at[hbase], ht.at[0], isem.at[0])
    pltpu.async_copy(ht_hbm.at[hbase + 1], ht.at[1], isem.at[1])
    pltpu.make_async_copy(ht_hbm.at[hbase], ht.at[0], isem.at[0]).wait()
    pltpu.async_copy(y0_hbm.at[ht.at[0, 1]], rows.at[0], gsem.at[0])

    def quad(j4, carry):
        for u in range(4):
            j = j4 * 4 + u
            b = u % 2
            u1 = (u + 1) % 4
            u2 = (u + 2) % 4
            pltpu.make_async_copy(
                y0_hbm.at[ht.at[u, 1]], rows.at[b], gsem.at[b]).wait()
            pltpu.async_copy(rows.at[b], xacc.at[ht.at[u, 0]], ssem.at[b],
                             add=True)

            @pl.when(j + 1 < CHUNKS)
            def _():
                pltpu.make_async_copy(
                    ht_hbm.at[hbase + j + 1], ht.at[u1], isem.at[u1]).wait()

                @pl.when(j >= 1)
                def _():
                    pltpu.make_async_copy(
                        rows.at[1 - b], xacc.at[ht.at[u1, 0]],
                        ssem.at[1 - b]).wait()

                pltpu.async_copy(
                    y0_hbm.at[ht.at[u1, 1]], rows.at[1 - b], gsem.at[1 - b])

            @pl.when(j + 2 < CHUNKS)
            def _():
                pltpu.async_copy(
                    ht_hbm.at[hbase + j + 2], ht.at[u2], isem.at[u2])
        return carry

    lax.fori_loop(0, CHUNKS // 4, quad, 0)
    for b in range(2):
        pltpu.make_async_copy(
            rows.at[b], xacc.at[ht.at[0, 0]], ssem.at[b]).wait()
    plsc.subcore_barrier()
    for q in range(ZQ):
        pltpu.sync_copy(xacc.at[pl.ds(s * ZR + q * CH, CH)], rows.at[0])
        pltpu.sync_copy(rows.at[0], out_hbm.at[c, pl.ds(s * ZR + q * CH, CH)])


@functools.partial(
    pl.kernel,
    out_type=(
        jax.ShapeDtypeStruct((NW, GC, F, G8, CH), jnp.float32),
        jax.ShapeDtypeStruct((NC, F, N_ACC), jnp.float32),
    ),
    mesh=_MESH,
    compiler_params=pltpu.CompilerParams(needs_layout_passes=False),
    scratch_types=[
        pltpu.VMEM((GC, G8, CH), jnp.int32),
        pltpu.VMEM((GC, G8, CH), jnp.int32),
        pltpu.VMEM((2, CH, D), jnp.float32),
        pltpu.VMEM((2, CH, D), jnp.float32),
        pltpu.VMEM((2, F, G8, CH), jnp.float32),
        pltpu.VMEM((ZR,), jnp.float32),
        pltpu.VMEM_SHARED((N_ACC,), jnp.float32),
        pltpu.VMEM_SHARED((N_ACC,), jnp.float32),
        pltpu.VMEM_SHARED((N_ACC,), jnp.float32),
        pltpu.VMEM_SHARED((N_ACC,), jnp.float32),
        pltpu.SemaphoreType.DMA((2,)),
        pltpu.SemaphoreType.DMA((2,)),
        pltpu.SemaphoreType.DMA((2,)),
    ],
)
def _sc_route(xn_hbm, tn_hbm, h3_hbm, t3_hbm, scores_hbm, rsum_hbm,
              h3, t3, ab, bb, sbw, stage, r0, r1, r2, r3,
              agsem, bgsem, wsem):
    c = lax.axis_index("c")
    s = lax.axis_index("s")
    w = _wid()
    raccs = (r0, r1, r2, r3)
    pltpu.sync_copy(h3_hbm.at[w], h3)
    pltpu.sync_copy(t3_hbm.at[w], t3)
    lane = lax.iota(jnp.int32, 16)
    sws = {sz: jnp.bitwise_xor(lane, sz) for sz in (8, 4, 2, 1)}
    m8 = jnp.bitwise_and(lane, 8) == 0
    m4 = jnp.bitwise_and(lane, 4) == 0
    m2 = jnp.bitwise_and(lane, 2) == 0
    m1 = jnp.bitwise_and(lane, 1) == 0

    def zrow(r, carry):
        stage[pl.ds(16 * r, 16)] = jnp.zeros((16,), jnp.float32)
        return carry

    lax.fori_loop(0, ZR // 16, zrow, 0)
    for i in range(F):
        pltpu.sync_copy(stage, raccs[i].at[pl.ds(s * ZR, ZR)])
    plsc.subcore_barrier()

    pltpu.async_copy(xn_hbm.at[h3.at[0, 0]], ab.at[0], agsem.at[0])
    pltpu.async_copy(tn_hbm.at[t3.at[0, 0]], bb.at[0], bgsem.at[0])

    def _drain_writes(bg):
        pltpu.make_async_copy(sbw.at[bg], scores_hbm.at[0, 0],
                              wsem.at[bg]).wait()

    def _one_group(jg, bg):
        @pl.when(jg >= 2)
        def _():
            _drain_writes(bg)

        def pairs(pp, carry):
            for uu in range(2):
                u = 2 * pp + uu
                j = jg * G8 + u
                b = uu
                pltpu.make_async_copy(
                    xn_hbm.at[h3.at[jg, u]], ab.at[b], agsem.at[b]).wait()
                pltpu.make_async_copy(
                    tn_hbm.at[t3.at[jg, u]], bb.at[b], bgsem.at[b]).wait()

                @pl.when(j + 1 < CHUNKS)
                def _():
                    jn = j + 1
                    jgn = jn // G8
                    un = jn % G8
                    pltpu.async_copy(
                        xn_hbm.at[h3.at[jgn, un]], ab.at[1 - b],
                        agsem.at[1 - b])
                    pltpu.async_copy(
                        tn_hbm.at[t3.at[jgn, un]], bb.at[1 - b],
                        bgsem.at[1 - b])

                def group(g, carry2):
                    dv = []
                    for i in range(F):
                        lvl = []
                        for k in range(16):
                            e = 16 * g + _BITREV[k]
                            a0 = ab[b, e, pl.ds(FD * i, 16)]
                            a1 = ab[b, e, pl.ds(FD * i + 16, 16)]
                            t0 = bb[b, e, pl.ds(FD * i, 16)]
                            t1 = bb[b, e, pl.ds(FD * i + 16, 16)]
                            lvl.append(a0 * t0 + a1 * t1)
                        for sz, msk in ((8, m8), (4, m4), (2, m2), (1, m1)):
                            sidx = sws[sz]
                            nxt = []
                            for k in range(len(lvl) // 2):
                                x, y = lvl[2 * k], lvl[2 * k + 1]
                                zx = x + jnp.take_along_axis(x, sidx, axis=0)
                                zy = y + jnp.take_along_axis(y, sidx, axis=0)
                                nxt.append(jnp.where(msk, zx, zy))
                            lvl = nxt
                        dv.append(lvl[0])
                    m = jnp.maximum(jnp.maximum(dv[0], dv[1]),
                                    jnp.maximum(dv[2], dv[3]))
                    ex = [jnp.exp(x - m) for x in dv]
                    inv = 1.0 / (ex[0] + ex[1] + ex[2] + ex[3])
                    for i in range(F):
                        sbw[bg, i, u, pl.ds(16 * g, 16)] = ex[i] * inv
                    return carry2

                lax.fori_loop(0, CH // 16, group, 0)
                for i in range(F):
                    pltpu.sync_copy(sbw.at[bg, i, u],
                                    raccs[i].at[h3.at[jg, u]], add=True)
            return carry

        lax.fori_loop(0, G8 // 2, pairs, 0)
        pltpu.async_copy(sbw.at[bg], scores_hbm.at[w, jg], wsem.at[bg])

    def gpair(p, carry):
        _one_group(2 * p, 0)
        _one_group(2 * p + 1, 1)
        return carry

    lax.fori_loop(0, GC // 2, gpair, 0)
    for bg in range(2):
        _drain_writes(bg)
    plsc.subcore_barrier()
    for i in range(F):
        pltpu.sync_copy(raccs[i].at[pl.ds(s * ZR, ZR)], stage)
        pltpu.sync_copy(stage, rsum_hbm.at[c, i, pl.ds(s * ZR, ZR)])


@functools.partial(
    pl.kernel,
    out_type=jax.ShapeDtypeStruct((NC, N_ACC, D), jnp.float32),
    mesh=_MESH,
    compiler_params=pltpu.CompilerParams(needs_layout_passes=False),
    scratch_types=[
        pltpu.VMEM((4, 2, CH), jnp.int32),
        pltpu.VMEM((2, CH, D), jnp.float32),
        pltpu.VMEM((2, F, CH), jnp.float32),
        pltpu.VMEM_SHARED((N_ACC, D), jnp.float32),
        pltpu.SemaphoreType.DMA((4,)),
        pltpu.SemaphoreType.DMA((2,)),
        pltpu.SemaphoreType.DMA((2,)),
        pltpu.SemaphoreType.DMA((2,)),
    ],
)
def _sc_pass1(y1_hbm, sc_hbm, ht_hbm, out_hbm,
              ht, rows, sb, xacc, isem, gsem, ssem, qsem):
    c = lax.axis_index("c")
    s = lax.axis_index("s")
    w = _wid()
    hbase = w * CHUNKS
    _zero_rows(rows.at[0])
    for q in range(ZQ):
        pltpu.sync_copy(rows.at[0], xacc.at[pl.ds(s * ZR + q * CH, CH)])
    plsc.subcore_barrier()

    pltpu.async_copy(ht_hbm.at[hbase], ht.at[0], isem.at[0])
    pltpu.async_copy(ht_hbm.at[hbase + 1], ht.at[1], isem.at[1])
    pltpu.async_copy(sc_hbm.at[w, 0, :, 0], sb.at[0], qsem.at[0])
    pltpu.async_copy(sc_hbm.at[w, 0, :, 1], sb.at[1], qsem.at[1])
    pltpu.make_async_copy(ht_hbm.at[hbase], ht.at[0], isem.at[0]).wait()
    pltpu.async_copy(y1_hbm.at[ht.at[0, 1]], rows.at[0], gsem.at[0])

    def quad(j4, carry):
        for u in range(4):
            j = j4 * 4 + u
            b = u % 2
            u1 = (u + 1) % 4
            u2 = (u + 2) % 4
            pltpu.make_async_copy(
                y1_hbm.at[ht.at[u, 1]], rows.at[b], gsem.at[b]).wait()
            pltpu.make_async_copy(
                sc_hbm.at[w, 0, :, 0], sb.at[b], qsem.at[b]).wait()

            def group(g, carry2):
                svs = [sb[b, i, pl.ds(16 * g, 16)] for i in range(F)]
                for uu in range(16):
                    e = 16 * g + uu
                    for i in range(F):
                        sv = svs[i][uu]
                        for k in (2 * i, 2 * i + 1):
                            rows[b, e, pl.ds(16 * k, 16)] = (
                                rows[b, e, pl.ds(16 * k, 16)] * sv)
                return carry2

            lax.fori_loop(0, CH // 16, group, 0)
            pltpu.async_copy(rows.at[b], xacc.at[ht.at[u, 0]], ssem.at[b],
                             add=True)

            @pl.when(j + 2 < CHUNKS)
            def _():
                jn = j + 2
                pltpu.async_copy(
                    sc_hbm.at[w, jn // G8, :, jn % G8], sb.at[b],
                    qsem.at[b])

            @pl.when(j + 1 < CHUNKS)
            def _():
                pltpu.make_async_copy(
                    ht_hbm.at[hbase + j + 1], ht.at[u1], isem.at[u1]).wait()

                @pl.when(j >= 1)
                def _():
                    pltpu.make_async_copy(
                        rows.at[1 - b], xacc.at[ht.at[u1, 0]],
                        ssem.at[1 - b]).wait()

                pltpu.async_copy(
                    y1_hbm.at[ht.at[u1, 1]], rows.at[1 - b], gsem.at[1 - b])

            @pl.when(j + 2 < CHUNKS)
            def _():
                pltpu.async_copy(
                    ht_hbm.at[hbase + j + 2], ht.at[u2], isem.at[u2])
        return carry

    lax.fori_loop(0, CHUNKS // 4, quad, 0)
    for b in range(2):
        pltpu.make_async_copy(
            rows.at[b], xacc.at[ht.at[0, 0]], ssem.at[b]).wait()
    plsc.subcore_barrier()
    for q in range(ZQ):
        pltpu.sync_copy(xacc.at[pl.ds(s * ZR + q * CH, CH)], rows.at[0])
        pltpu.sync_copy(rows.at[0], out_hbm.at[c, pl.ds(s * ZR + q * CH, CH)])


def _slicenorm(x):
    outs = []
    for i in range(F):
        sl = x[:, i * FD:(i + 1) * FD]
        n = jnp.sqrt(jnp.sum(sl * sl, axis=1, keepdims=True))
        outs.append(sl / jnp.maximum(n, 1e-12))
    return jnp.concatenate(outs, axis=1)


def _tc_prep_body(ego_ref, deg_ref, y0_ref, tn_ref):
    ego = ego_ref[...]
    d0 = lax.rsqrt(0.25 * deg_ref[...] + 1e-12)
    y0_ref[...] = ego * d0
    tn_ref[...] = _slicenorm(jnp.tanh(ego))


def _tc_mid0_body(x0p_ref, xn_ref):
    r = x0p_ref[0] + x0p_ref[1]
    rowid = lax.broadcasted_iota(jnp.int32, (N_ACC, D), 0)
    r = jnp.where(rowid < N, r, 0.0)
    xn_ref[...] = _slicenorm(r)


def _tc_y1_body(ego_ref, rse_ref, y1_ref, d1e_ref):
    d1e = lax.rsqrt(rse_ref[...] + 1e-12)
    d1e_ref[...] = d1e
    y1_ref[...] = ego_ref[...] * d1e


def _tc_final_body(ego_ref, x1p_ref, d1e_ref, out_ref):
    x1 = (x1p_ref[0] + x1p_ref[1]) * d1e_ref[...]
    out_ref[...] = 0.5 * (ego_ref[...] + x1)


_f32 = jnp.float32


@jax.jit
def kernel(user_embedding, item_embedding, all_h_list, all_t_list):
    ego = jnp.concatenate(
        [user_embedding, item_embedding,
         jnp.zeros((N_ACC - N, D), _f32)], axis=0)
    pad = jnp.full((E_PAD - E,), N, jnp.int32)
    h2 = jnp.concatenate([all_h_list.astype(jnp.int32), pad]).reshape(-1, CH)
    t2 = jnp.concatenate([all_t_list.astype(jnp.int32), pad]).reshape(-1, CH)
    ht2 = jnp.stack([h2, t2], axis=1)
    h3 = h2.reshape(NW, GC, G8, CH)
    t3 = t2.reshape(NW, GC, G8, CH)
    degp = _sc_deg(h2)
    deg = (degp[0] + degp[1])[:, None]

    y0, tn = pl.pallas_call(
        _tc_prep_body,
        out_shape=(jax.ShapeDtypeStruct((N_ACC, D), _f32),
                   jax.ShapeDtypeStruct((N_ACC, D), _f32)))(ego, deg)

    x0p = _sc_pass0(y0, ht2)

    xn0 = pl.pallas_call(
        _tc_mid0_body,
        out_shape=jax.ShapeDtypeStruct((N_ACC, D), _f32))(x0p)

    scores, rsump = _sc_route(xn0, tn, h3, t3)
    rs = rsump[0] + rsump[1]
    rse = jnp.repeat(rs.T, FD, axis=1)

    y1, d1e = pl.pallas_call(
        _tc_y1_body,
        out_shape=(jax.ShapeDtypeStruct((N_ACC, D), _f32),
                   jax.ShapeDtypeStruct((N_ACC, D), _f32)))(ego, rse)

    x1p = _sc_pass1(y1, scores, ht2)

    out = pl.pallas_call(
        _tc_final_body,
        out_shape=jax.ShapeDtypeStruct((N_ACC, D), _f32))(ego, x1p, d1e)
    return out[:N_U], out[N_U:N]


# speedup vs baseline: 8.7067x; 1.0788x over previous
import functools
import jax
import jax.numpy as jnp
from jax import lax
from jax.experimental import pallas as pl
from jax.experimental.pallas import tpu as pltpu
from jax.experimental.pallas import tpu_sc as plsc

N_U = 4000
N_I = 6000
N = N_U + N_I
E = 320000
D = 128
F = 4
FD = D // F

NC, NS = 2, 16
NW = NC * NS
CH = 128
EPT = 10240
CHUNKS = EPT // CH
E_PAD = NW * EPT
G8 = 8
GC = CHUNKS // G8
N_ACC = 10240
ZR = N_ACC // NS
ZQ = ZR // CH

_BITREV = (0, 8, 4, 12, 2, 10, 6, 14, 1, 9, 5, 13, 3, 11, 7, 15)

_MESH = plsc.VectorSubcoreMesh(
    core_axis_name="c", subcore_axis_name="s", num_cores=NC, num_subcores=NS)


def _wid():
    return lax.axis_index("c") * NS + lax.axis_index("s")


def _zero_rows(rows):
    def zrow(r, carry):
        for k in range(D // 16):
            rows[r, pl.ds(16 * k, 16)] = jnp.zeros((16,), jnp.float32)
        return carry

    lax.fori_loop(0, CH, zrow, 0)


@functools.partial(
    pl.kernel,
    out_type=jax.ShapeDtypeStruct((NC, N_ACC), jnp.float32),
    mesh=_MESH,
    compiler_params=pltpu.CompilerParams(needs_layout_passes=False),
    scratch_types=[
        pltpu.VMEM((CHUNKS, CH), jnp.int32),
        pltpu.VMEM((CH,), jnp.float32),
        pltpu.VMEM((ZR,), jnp.float32),
        pltpu.VMEM_SHARED((N_ACC,), jnp.float32),
    ],
)
def _sc_deg(h2_hbm, out_hbm, hidx, ones, stage, acc):
    c = lax.axis_index("c")
    s = lax.axis_index("s")
    w = _wid()
    pltpu.sync_copy(h2_hbm.at[pl.ds(w * CHUNKS, CHUNKS)], hidx)
    for i in range(CH // 16):
        ones[pl.ds(16 * i, 16)] = jnp.ones((16,), jnp.float32)

    def zrow(r, carry):
        stage[pl.ds(16 * r, 16)] = jnp.zeros((16,), jnp.float32)
        return carry

    lax.fori_loop(0, ZR // 16, zrow, 0)
    pltpu.sync_copy(stage, acc.at[pl.ds(s * ZR, ZR)])
    plsc.subcore_barrier()

    def chunk(j, carry):
        pltpu.sync_copy(ones, acc.at[hidx.at[j]], add=True)
        return carry

    lax.fori_loop(0, CHUNKS, chunk, 0)
    plsc.subcore_barrier()
    pltpu.sync_copy(acc.at[pl.ds(s * ZR, ZR)], stage)
    pltpu.sync_copy(stage, out_hbm.at[c, pl.ds(s * ZR, ZR)])


@functools.partial(
    pl.kernel,
    out_type=jax.ShapeDtypeStruct((NC, N_ACC, D), jnp.float32),
    mesh=_MESH,
    compiler_params=pltpu.CompilerParams(needs_layout_passes=False),
    scratch_types=[
        pltpu.VMEM((4, 2, CH), jnp.int32),
        pltpu.VMEM((2, CH, D), jnp.float32),
        pltpu.VMEM_SHARED((N_ACC, D), jnp.float32),
        pltpu.SemaphoreType.DMA((4,)),
        pltpu.SemaphoreType.DMA((2,)),
        pltpu.SemaphoreType.DMA((2,)),
    ],
)
def _sc_pass0(y0_hbm, ht_hbm, out_hbm, ht, rows, xacc, isem, gsem, ssem):
    c = lax.axis_index("c")
    s = lax.axis_index("s")
    w = _wid()
    hbase = w * CHUNKS
    _zero_rows(rows.at[0])
    for q in range(ZQ):
        pltpu.sync_copy(rows.at[0], xacc.at[pl.ds(s * ZR + q * CH, CH)])
    plsc.subcore_barrier()

    pltpu.async_copy(ht_hbm.at[hbase], ht.at[0], isem.at[0])
    pltpu.async_copy(ht_hbm.at[hbase + 1], ht.at[1], isem.at[1])
    pltpu.make_async_copy(ht_hbm.at[hbase], ht.at[0], isem.at[0]).wait()
    pltpu.async_copy(y0_hbm.at[ht.at[0, 1]], rows.at[0], gsem.at[0])

    def quad(j4, carry):
        for u in range(4):
            j = j4 * 4 + u
            b = u % 2
            u1 = (u + 1) % 4
            u2 = (u + 2) % 4
            pltpu.make_async_copy(
                y0_hbm.at[ht.at[u, 1]], rows.at[b], gsem.at[b]).wait()
            pltpu.async_copy(rows.at[b], xacc.at[ht.at[u, 0]], ssem.at[b],
                             add=True)

            @pl.when(j + 1 < CHUNKS)
            def _():
                pltpu.make_async_copy(
                    ht_hbm.at[hbase + j + 1], ht.at[u1], isem.at[u1]).wait()

                @pl.when(j >= 1)
                def _():
                    pltpu.make_async_copy(
                        rows.at[1 - b], xacc.at[ht.at[u1, 0]],
                        ssem.at[1 - b]).wait()

                pltpu.async_copy(
                    y0_hbm.at[ht.at[u1, 1]], rows.at[1 - b], gsem.at[1 - b])

            @pl.when(j + 2 < CHUNKS)
            def _():
                pltpu.async_copy(
                    ht_hbm.at[hbase + j + 2], ht.at[u2], isem.at[u2])
        return carry

    lax.fori_loop(0, CHUNKS // 4, quad, 0)
    for b in range(2):
        pltpu.make_async_copy(
            rows.at[b], xacc.at[ht.at[0, 0]], ssem.at[b]).wait()
    plsc.subcore_barrier()
    for q in range(ZQ):
        pltpu.sync_copy(xacc.at[pl.ds(s * ZR + q * CH, CH)], rows.at[0])
        pltpu.sync_copy(rows.at[0], out_hbm.at[c, pl.ds(s * ZR + q * CH, CH)])


@functools.partial(
    pl.kernel,
    out_type=(
        jax.ShapeDtypeStruct((NW, GC, G8, F * CH), jnp.float32),
        jax.ShapeDtypeStruct((NC, F * N_ACC), jnp.float32),
    ),
    mesh=_MESH,
    compiler_params=pltpu.CompilerParams(needs_layout_passes=False),
    scratch_types=[
        pltpu.VMEM((GC, G8, CH), jnp.int32),
        pltpu.VMEM((GC, G8, CH), jnp.int32),
        pltpu.VMEM((2, CH, D), jnp.float32),
        pltpu.VMEM((2, CH, D), jnp.float32),
        pltpu.VMEM((2, G8, F * CH), jnp.float32),
        pltpu.VMEM((F * CH,), jnp.int32),
        pltpu.VMEM((F * CH,), jnp.float32),
        pltpu.VMEM((ZR,), jnp.float32),
        pltpu.VMEM_SHARED((F * N_ACC,), jnp.float32),
        pltpu.SemaphoreType.DMA((2,)),
        pltpu.SemaphoreType.DMA((2,)),
        pltpu.SemaphoreType.DMA((2,)),
    ],
)
def _sc_route(xn_hbm, tn_hbm, h3_hbm, t3_hbm, scores_hbm, rsum_hbm,
              h3, t3, ab, bb, sbw, oidx, sval, stage, racc,
              agsem, bgsem, wsem):
    c = lax.axis_index("c")
    s = lax.axis_index("s")
    w = _wid()
    pltpu.sync_copy(h3_hbm.at[w], h3)
    pltpu.sync_copy(t3_hbm.at[w], t3)
    lane = lax.iota(jnp.int32, 16)
    sws = {sz: jnp.bitwise_xor(lane, sz) for sz in (8, 4, 2, 1)}
    m8 = jnp.bitwise_and(lane, 8) == 0
    m4 = jnp.bitwise_and(lane, 4) == 0
    m2 = jnp.bitwise_and(lane, 2) == 0
    m1 = jnp.bitwise_and(lane, 1) == 0

    def zrow(r, carry):
        stage[pl.ds(16 * r, 16)] = jnp.zeros((16,), jnp.float32)
        return carry

    lax.fori_loop(0, ZR // 16, zrow, 0)
    for q in range(F):
        pltpu.sync_copy(stage, racc.at[pl.ds((q * NS + s) * ZR, ZR)])
    plsc.subcore_barrier()

    pltpu.async_copy(xn_hbm.at[h3.at[0, 0]], ab.at[0], agsem.at[0])
    pltpu.async_copy(tn_hbm.at[t3.at[0, 0]], bb.at[0], bgsem.at[0])

    def _drain_writes(bg):
        pltpu.make_async_copy(sbw.at[bg], scores_hbm.at[0, 0],
                              wsem.at[bg]).wait()


    def _one_group(jg, bg):
        @pl.when(jg >= 2)
        def _():
            _drain_writes(bg)

        def pairs(pp, carry):
            for uu in range(2):
                u = 2 * pp + uu
                j = jg * G8 + u
                b = uu
                pltpu.make_async_copy(
                    xn_hbm.at[h3.at[jg, u]], ab.at[b], agsem.at[b]).wait()
                pltpu.make_async_copy(
                    tn_hbm.at[t3.at[jg, u]], bb.at[b], bgsem.at[b]).wait()

                @pl.when(j + 1 < CHUNKS)
                def _():
                    jn = j + 1
                    jgn = jn // G8
                    un = jn % G8
                    pltpu.async_copy(
                        xn_hbm.at[h3.at[jgn, un]], ab.at[1 - b],
                        agsem.at[1 - b])
                    pltpu.async_copy(
                        tn_hbm.at[t3.at[jgn, un]], bb.at[1 - b],
                        bgsem.at[1 - b])

                def group(g, carry2):
                    dv = []
                    for i in range(F):
                        lvl = []
                        for k in range(16):
                            e = 16 * g + _BITREV[k]
                            a0 = ab[b, e, pl.ds(FD * i, 16)]
                            a1 = ab[b, e, pl.ds(FD * i + 16, 16)]
                            t0 = bb[b, e, pl.ds(FD * i, 16)]
                            t1 = bb[b, e, pl.ds(FD * i + 16, 16)]
                            lvl.append(a0 * t0 + a1 * t1)
                        for sz, msk in ((8, m8), (4, m4), (2, m2), (1, m1)):
                            sidx = sws[sz]
                            nxt = []
                            for k in range(len(lvl) // 2):
                                x, y = lvl[2 * k], lvl[2 * k + 1]
                                zx = x + jnp.take_along_axis(x, sidx, axis=0)
                                zy = y + jnp.take_along_axis(y, sidx, axis=0)
                                nxt.append(jnp.where(msk, zx, zy))
                            lvl = nxt
                        dv.append(lvl[0])
                    m = jnp.maximum(jnp.maximum(dv[0], dv[1]),
                                    jnp.maximum(dv[2], dv[3]))
                    ex = [jnp.exp(x - m) for x in dv]
                    inv = 1.0 / (ex[0] + ex[1] + ex[2] + ex[3])
                    for i in range(F):
                        sc = ex[i] * inv
                        sbw[bg, u, pl.ds(i * CH + 16 * g, 16)] = sc
                        sval[pl.ds(i * CH + 16 * g, 16)] = sc
                    return carry2

                lax.fori_loop(0, CH // 16, group, 0)
                for v in range(CH // 16):
                    hv = h3[jg, u, pl.ds(16 * v, 16)]
                    for i in range(F):
                        oidx[pl.ds(i * CH + 16 * v, 16)] = hv + (i * N_ACC)
                pltpu.sync_copy(sval, racc.at[oidx], add=True)
            return carry

        lax.fori_loop(0, G8 // 2, pairs, 0)
        pltpu.async_copy(sbw.at[bg], scores_hbm.at[w, jg], wsem.at[bg])

    def gpair(p, carry):
        _one_group(2 * p, 0)
        _one_group(2 * p + 1, 1)
        return carry

    lax.fori_loop(0, GC // 2, gpair, 0)
    for bg in range(2):
        _drain_writes(bg)
    plsc.subcore_barrier()
    for q in range(F):
        pltpu.sync_copy(racc.at[pl.ds((q * NS + s) * ZR, ZR)], stage)
        pltpu.sync_copy(stage, rsum_hbm.at[c, pl.ds((q * NS + s) * ZR, ZR)])


@functools.partial(
    pl.kernel,
    out_type=jax.ShapeDtypeStruct((NC, N_ACC, D), jnp.float32),
    mesh=_MESH,
    compiler_params=pltpu.CompilerParams(needs_layout_passes=False),
    scratch_types=[
        pltpu.VMEM((4, 2, CH), jnp.int32),
        pltpu.VMEM((2, CH, D), jnp.float32),
        pltpu.VMEM((2, F * CH), jnp.float32),
        pltpu.VMEM_SHARED((N_ACC, D), jnp.float32),
        pltpu.SemaphoreType.DMA((4,)),
        pltpu.SemaphoreType.DMA((2,)),
        pltpu.SemaphoreType.DMA((2,)),
        pltpu.SemaphoreType.DMA((2,)),
    ],
)
def _sc_pass1(y1_hbm, sc_hbm, ht_hbm, out_hbm,
              ht, rows, sb, xacc, isem, gsem, ssem, qsem):
    c = lax.axis_index("c")
    s = lax.axis_index("s")
    w = _wid()
    hbase = w * CHUNKS
    _zero_rows(rows.at[0])
    for q in range(ZQ):
        pltpu.sync_copy(rows.at[0], xacc.at[pl.ds(s * ZR + q * CH, CH)])
    plsc.subcore_barrier()

    pltpu.async_copy(ht_hbm.at[hbase], ht.at[0], isem.at[0])
    pltpu.async_copy(ht_hbm.at[hbase + 1], ht.at[1], isem.at[1])
    pltpu.async_copy(sc_hbm.at[w, 0, 0], sb.at[0], qsem.at[0])
    pltpu.async_copy(sc_hbm.at[w, 0, 1], sb.at[1], qsem.at[1])
    pltpu.make_async_copy(ht_hbm.at[hbase], ht.at[0], isem.at[0]).wait()
    pltpu.async_copy(y1_hbm.at[ht.at[0, 1]], rows.at[0], gsem.at[0])

    def quad(j4, carry):
        for u in range(4):
            j = j4 * 4 + u
            b = u % 2
            u1 = (u + 1) % 4
            u2 = (u + 2) % 4
            pltpu.make_async_copy(
                y1_hbm.at[ht.at[u, 1]], rows.at[b], gsem.at[b]).wait()
            pltpu.make_async_copy(
                sc_hbm.at[w, 0, 0], sb.at[b], qsem.at[b]).wait()

            def group(g, carry2):
                svs = [sb[b, pl.ds(i * CH + 16 * g, 16)]
                       for i in range(F)]
                for uu in range(16):
                    e = 16 * g + uu
                    for i in range(F):
                        sv = svs[i][uu]
                        for k in (2 * i, 2 * i + 1):
                            rows[b, e, pl.ds(16 * k, 16)] = (
                                rows[b, e, pl.ds(16 * k, 16)] * sv)
                return carry2

            lax.fori_loop(0, CH // 16, group, 0)
            pltpu.async_copy(rows.at[b], xacc.at[ht.at[u, 0]], ssem.at[b],
                             add=True)

            @pl.when(j + 2 < CHUNKS)
            def _():
                jn = j + 2
                pltpu.async_copy(
                    sc_hbm.at[w, jn // G8, jn % G8], sb.at[b],
                    qsem.at[b])

            @pl.when(j + 1 < CHUNKS)
            def _():
                pltpu.make_async_copy(
                    ht_hbm.at[hbase + j + 1], ht.at[u1], isem.at[u1]).wait()

                @pl.when(j >= 1)
                def _():
                    pltpu.make_async_copy(
                        rows.at[1 - b], xacc.at[ht.at[u1, 0]],
                        ssem.at[1 - b]).wait()

                pltpu.async_copy(
                    y1_hbm.at[ht.at[u1, 1]], rows.at[1 - b], gsem.at[1 - b])

            @pl.when(j + 2 < CHUNKS)
            def _():
                pltpu.async_copy(
                    ht_hbm.at[hbase + j + 2], ht.at[u2], isem.at[u2])
        return carry

    lax.fori_loop(0, CHUNKS // 4, quad, 0)
    for b in range(2):
        pltpu.make_async_copy(
            rows.at[b], xacc.at[ht.at[0, 0]], ssem.at[b]).wait()
    plsc.subcore_barrier()
    for q in range(ZQ):
        pltpu.sync_copy(xacc.at[pl.ds(s * ZR + q * CH, CH)], rows.at[0])
        pltpu.sync_copy(rows.at[0], out_hbm.at[c, pl.ds(s * ZR + q * CH, CH)])


def _slicenorm(x):
    outs = []
    for i in range(F):
        sl = x[:, i * FD:(i + 1) * FD]
        n = jnp.sqrt(jnp.sum(sl * sl, axis=1, keepdims=True))
        outs.append(sl / jnp.maximum(n, 1e-12))
    return jnp.concatenate(outs, axis=1)


def _tc_prep_body(ego_ref, deg_ref, y0_ref, tn_ref):
    ego = ego_ref[...]
    d0 = lax.rsqrt(0.25 * deg_ref[...] + 1e-12)
    y0_ref[...] = ego * d0
    tn_ref[...] = _slicenorm(jnp.tanh(ego))


def _tc_mid0_body(x0p_ref, xn_ref):
    r = x0p_ref[0] + x0p_ref[1]
    rowid = lax.broadcasted_iota(jnp.int32, (N_ACC, D), 0)
    r = jnp.where(rowid < N, r, 0.0)
    xn_ref[...] = _slicenorm(r)


def _tc_y1_body(ego_ref, rse_ref, y1_ref, d1e_ref):
    d1e = lax.rsqrt(rse_ref[...] + 1e-12)
    d1e_ref[...] = d1e
    y1_ref[...] = ego_ref[...] * d1e


def _tc_final_body(ego_ref, x1p_ref, d1e_ref, out_ref):
    x1 = (x1p_ref[0] + x1p_ref[1]) * d1e_ref[...]
    out_ref[...] = 0.5 * (ego_ref[...] + x1)


_f32 = jnp.float32


@jax.jit
def kernel(user_embedding, item_embedding, all_h_list, all_t_list):
    ego = jnp.concatenate(
        [user_embedding, item_embedding,
         jnp.zeros((N_ACC - N, D), _f32)], axis=0)
    pad = jnp.full((E_PAD - E,), N, jnp.int32)
    h2 = jnp.concatenate([all_h_list.astype(jnp.int32), pad]).reshape(-1, CH)
    t2 = jnp.concatenate([all_t_list.astype(jnp.int32), pad]).reshape(-1, CH)
    ht2 = jnp.stack([h2, t2], axis=1)
    h3 = h2.reshape(NW, GC, G8, CH)
    t3 = t2.reshape(NW, GC, G8, CH)
    degp = _sc_deg(h2)
    deg = (degp[0] + degp[1])[:, None]

    y0, tn = pl.pallas_call(
        _tc_prep_body,
        out_shape=(jax.ShapeDtypeStruct((N_ACC, D), _f32),
                   jax.ShapeDtypeStruct((N_ACC, D), _f32)))(ego, deg)

    x0p = _sc_pass0(y0, ht2)

    xn0 = pl.pallas_call(
        _tc_mid0_body,
        out_shape=jax.ShapeDtypeStruct((N_ACC, D), _f32))(x0p)

    scores, rsump = _sc_route(xn0, tn, h3, t3)
    rs = (rsump[0] + rsump[1]).reshape(F, N_ACC)
    rse = jnp.repeat(rs.T, FD, axis=1)

    y1, d1e = pl.pallas_call(
        _tc_y1_body,
        out_shape=(jax.ShapeDtypeStruct((N_ACC, D), _f32),
                   jax.ShapeDtypeStruct((N_ACC, D), _f32)))(ego, rse)

    x1p = _sc_pass1(y1, scores, ht2)

    out = pl.pallas_call(
        _tc_final_body,
        out_shape=jax.ShapeDtypeStruct((N_ACC, D), _f32))(ego, x1p, d1e)
    return out[:N_U], out[N_U:N]
